# baseline (speedup 1.0000x reference)
import jax
import jax.numpy as jnp
from jax.experimental import pallas as pl
from jax.experimental.pallas import tpu as pltpu

_IN_C = 192
_OUT_C = 384
_W = 0.5


def _scale_copy(dest_ref, x_ref, o_ref):
    o_ref[...] = x_ref[...] * _W


def kernel(x, perm):
    B, C, H, W = x.shape
    HW = H * W
    S = HW // 128
    xr = x.reshape(B, 1, C, S, 128)

    perm32 = perm.astype(jnp.int32)
    ar = jnp.arange(_IN_C, dtype=jnp.int32)
    z = jnp.zeros((_IN_C,), jnp.int32)
    dest0 = z.at[perm32[:_IN_C]].set(ar)
    dest1 = z.at[perm32[_IN_C:]].set(ar)
    dests = jnp.stack([dest0, dest1])

    out = pl.pallas_call(
        _scale_copy,
        grid_spec=pltpu.PrefetchScalarGridSpec(
            num_scalar_prefetch=1,
            grid=(_IN_C, 2),
            in_specs=[
                pl.BlockSpec(
                    (B, 1, 1, S, 128), lambda i, h, dest_ref: (0, 0, i, 0, 0)
                )
            ],
            out_specs=pl.BlockSpec(
                (B, 1, 1, S, 128),
                lambda i, h, dest_ref: (0, h, dest_ref[h, i], 0, 0),
            ),
        ),
        out_shape=jax.ShapeDtypeStruct((B, 2, _IN_C, S, 128), x.dtype),
    )(dests, xr)
    return out.reshape(B, _OUT_C, H, W)


# speedup vs baseline: 1.2784x; 1.2784x over previous
import jax
import jax.numpy as jnp
from jax.experimental import pallas as pl
from jax.experimental.pallas import tpu as pltpu

_IN_C = 192
_OUT_C = 384
_W = 0.5
_NSLOT = 4


def _body(dest_ref, x_ref, out_ref, scratch, sem):
    b = pl.program_id(0)
    i = pl.program_id(1)
    step = b * _IN_C + i
    slot = jax.lax.rem(step, _NSLOT)

    def _copies(st, s):
        bb = jax.lax.div(st, _IN_C)
        ii = jax.lax.rem(st, _IN_C)
        d0 = dest_ref[ii]
        d1 = dest_ref[_IN_C + ii]
        return [
            pltpu.make_async_copy(
                scratch.at[s],
                out_ref.at[pl.ds(bb, 1), pl.ds(d, 1)],
                sem.at[s, k],
            )
            for k, d in enumerate((d0, d1))
        ]

    @pl.when(step >= _NSLOT)
    def _():
        for c in _copies(step - _NSLOT, slot):
            c.wait()

    scratch[slot] = x_ref[...] * _W

    for c in _copies(step, slot):
        c.start()

    @pl.when(step == 4 * _IN_C - 1)
    def _():
        for back in range(_NSLOT - 1, -1, -1):
            for c in _copies(step - back, jax.lax.rem(step - back, _NSLOT)):
                c.wait()


def kernel(x, perm):
    B, C, H, W = x.shape
    HW = H * W
    S = HW // 128
    xr = x.reshape(B, C, S, 128)

    perm32 = perm.astype(jnp.int32)
    ar = jnp.arange(_IN_C, dtype=jnp.int32)
    z = jnp.zeros((_IN_C,), jnp.int32)
    dest0 = z.at[perm32[:_IN_C]].set(ar)
    dest1 = z.at[perm32[_IN_C:]].set(ar) + _IN_C
    dests = jnp.concatenate([dest0, dest1])

    out = pl.pallas_call(
        _body,
        grid_spec=pltpu.PrefetchScalarGridSpec(
            num_scalar_prefetch=1,
            grid=(B, _IN_C),
            in_specs=[
                pl.BlockSpec((1, 1, S, 128), lambda b, i, dest_ref: (b, i, 0, 0))
            ],
            out_specs=pl.BlockSpec(memory_space=pl.MemorySpace.ANY),
            scratch_shapes=[
                pltpu.VMEM((_NSLOT, 1, 1, S, 128), jnp.float32),
                pltpu.SemaphoreType.DMA((_NSLOT, 2)),
            ],
        ),
        out_shape=jax.ShapeDtypeStruct((B, _OUT_C, S, 128), x.dtype),
    )(dests, xr)
    return out.reshape(B, _OUT_C, H, W)


# speedup vs baseline: 1.8630x; 1.4573x over previous
import jax
import jax.numpy as jnp
from jax.experimental import pallas as pl
from jax.experimental.pallas import tpu as pltpu

_IN_C = 192
_OUT_C = 384
_W = 0.5
_NSLOT = 4
_CPB = 2
_STEPS = _IN_C // _CPB


def _body(dest_ref, x_ref, out_ref, scratch, sem):
    i = pl.program_id(0)
    slot = jax.lax.rem(i, _NSLOT)

    def _copies(st, s):
        cs = []
        for k in range(_CPB):
            ch = st * _CPB + k
            for half in range(2):
                d = dest_ref[half * _IN_C + ch]
                cs.append(
                    pltpu.make_async_copy(
                        scratch.at[s, :, pl.ds(k, 1)],
                        out_ref.at[:, pl.ds(d, 1)],
                        sem.at[s, 2 * k + half],
                    )
                )
        return cs

    @pl.when(i >= _NSLOT)
    def _():
        for c in _copies(i - _NSLOT, slot):
            c.wait()

    scratch[slot] = x_ref[...] * _W

    for c in _copies(i, slot):
        c.start()

    @pl.when(i == _STEPS - 1)
    def _():
        for back in range(_NSLOT - 1, -1, -1):
            for c in _copies(i - back, jax.lax.rem(i - back, _NSLOT)):
                c.wait()


def kernel(x, perm):
    B, C, H, W = x.shape
    HW = H * W
    S = HW // 128
    xr = x.reshape(B, C, S, 128)

    perm32 = perm.astype(jnp.int32)
    ar = jnp.arange(_IN_C, dtype=jnp.int32)
    z = jnp.zeros((_IN_C,), jnp.int32)
    dest0 = z.at[perm32[:_IN_C]].set(ar)
    dest1 = z.at[perm32[_IN_C:]].set(ar) + _IN_C
    dests = jnp.concatenate([dest0, dest1])

    out = pl.pallas_call(
        _body,
        grid_spec=pltpu.PrefetchScalarGridSpec(
            num_scalar_prefetch=1,
            grid=(_STEPS,),
            in_specs=[
                pl.BlockSpec(
                    (B, _CPB, S, 128), lambda i, dest_ref: (0, i, 0, 0)
                )
            ],
            out_specs=pl.BlockSpec(memory_space=pl.MemorySpace.ANY),
            scratch_shapes=[
                pltpu.VMEM((_NSLOT, B, _CPB, S, 128), jnp.float32),
                pltpu.SemaphoreType.DMA((_NSLOT, 2 * _CPB)),
            ],
        ),
        out_shape=jax.ShapeDtypeStruct((B, _OUT_C, S, 128), x.dtype),
    )(dests, xr)
    return out.reshape(B, _OUT_C, H, W)


# speedup vs baseline: 1.8784x; 1.0083x over previous
import jax
import jax.numpy as jnp
from jax.experimental import pallas as pl
from jax.experimental.pallas import tpu as pltpu

_IN_C = 192
_OUT_C = 384
_W = 0.5
_NSLOT = 2
_CPB = 4
_STEPS = _IN_C // _CPB


def _body(dest_ref, x_ref, out_ref, scratch, sem):
    i = pl.program_id(0)
    slot = jax.lax.rem(i, _NSLOT)

    def _copies(st, s):
        cs = []
        for k in range(_CPB):
            ch = st * _CPB + k
            for half in range(2):
                d = dest_ref[half * _IN_C + ch]
                cs.append(
                    pltpu.make_async_copy(
                        scratch.at[s, :, pl.ds(k, 1)],
                        out_ref.at[:, pl.ds(d, 1)],
                        sem.at[s, 2 * k + half],
                    )
                )
        return cs

    @pl.when(i >= _NSLOT)
    def _():
        for c in _copies(i - _NSLOT, slot):
            c.wait()

    scratch[slot] = x_ref[...] * _W

    for c in _copies(i, slot):
        c.start()

    @pl.when(i == _STEPS - 1)
    def _():
        for back in range(_NSLOT - 1, -1, -1):
            for c in _copies(i - back, jax.lax.rem(i - back, _NSLOT)):
                c.wait()


def kernel(x, perm):
    B, C, H, W = x.shape
    HW = H * W
    S = HW // 128
    xr = x.reshape(B, C, S, 128)

    perm32 = perm.astype(jnp.int32)
    ar = jnp.arange(_IN_C, dtype=jnp.int32)
    z = jnp.zeros((_IN_C,), jnp.int32)
    dest0 = z.at[perm32[:_IN_C]].set(ar)
    dest1 = z.at[perm32[_IN_C:]].set(ar) + _IN_C
    dests = jnp.concatenate([dest0, dest1])

    out = pl.pallas_call(
        _body,
        grid_spec=pltpu.PrefetchScalarGridSpec(
            num_scalar_prefetch=1,
            grid=(_STEPS,),
            in_specs=[
                pl.BlockSpec(
                    (B, _CPB, S, 128), lambda i, dest_ref: (0, i, 0, 0)
                )
            ],
            out_specs=pl.BlockSpec(memory_space=pl.MemorySpace.ANY),
            scratch_shapes=[
                pltpu.VMEM((_NSLOT, B, _CPB, S, 128), jnp.float32),
                pltpu.SemaphoreType.DMA((_NSLOT, 2 * _CPB)),
            ],
        ),
        out_shape=jax.ShapeDtypeStruct((B, _OUT_C, S, 128), x.dtype),
    )(dests, xr)
    return out.reshape(B, _OUT_C, H, W)
